# Initial kernel scaffold; baseline (speedup 1.0000x reference)
import jax
import jax.numpy as jnp
from jax.experimental import pallas as pl


def kernel(visit_concept_orders, pe):
    raise NotImplementedError("write your pallas kernel here")


# speedup vs baseline: 3.3146x; 3.3146x over previous
import functools

import jax
import jax.numpy as jnp
from jax import lax
from jax.experimental import pallas as pl
from jax.experimental.pallas import tpu as pltpu
from jax.experimental.pallas import tpu_sc as plsc

LARGE_POSITION_VALUE = 1000000
MAX_SEQ_LEN = 512
EMB = 64
B, L = 4096, 200
NW = 32
RPW = B // NW
FLAT_PW = RPW * L
CHUNK = 128
NCHUNK = FLAT_PW // CHUNK
GROUPS = RPW // 16


def _body(vco_hbm, pe_hbm, out_hbm, vco_v, orders_v, rbuf0, rbuf1, sem0, sem1):
    cid = lax.axis_index("c")
    sid = lax.axis_index("s")
    wid = sid * 2 + cid
    row0 = wid * RPW
    out0 = wid * FLAT_PW

    pltpu.sync_copy(vco_hbm.at[pl.ds(wid * FLAT_PW, FLAT_PW)], vco_v)

    lanes = lax.broadcasted_iota(jnp.int32, (16,), 0)
    for g in range(GROUPS):
        rows = lanes + g * 16
        flat_base = rows * L

        def min_step(l, m):
            col = plsc.load_gather(vco_v, [flat_base + l])
            return jnp.minimum(m, jnp.where(col > 0, col, LARGE_POSITION_VALUE))

        m = lax.fori_loop(0, L, min_step, jnp.full((16,), LARGE_POSITION_VALUE, jnp.int32))

        def ord_step(l, carry):
            col = plsc.load_gather(vco_v, [flat_base + l])
            o = jnp.minimum(jnp.maximum(col - m, 0), jnp.int32(MAX_SEQ_LEN - 1))
            plsc.store_scatter(orders_v, [flat_base + l], o)
            return carry

        lax.fori_loop(0, L, ord_step, jnp.int32(0))

    def gather(c, buf, sem):
        return pltpu.async_copy(pe_hbm.at[orders_v.at[pl.ds(c * CHUNK, CHUNK)]], buf, sem)

    cp = gather(0, rbuf0, sem0)

    def chunk_step(i, carry):
        c = i * 2

        @pl.when(c + 1 < NCHUNK)
        def _():
            gather(c + 1, rbuf1, sem1)

        pltpu.make_async_copy(pe_hbm.at[orders_v.at[pl.ds(0, CHUNK)]], rbuf0, sem0).wait()
        pltpu.sync_copy(rbuf0, out_hbm.at[pl.ds(out0 + c * CHUNK, CHUNK)])

        @pl.when(c + 2 < NCHUNK)
        def _():
            gather(c + 2, rbuf0, sem0)

        @pl.when(c + 1 < NCHUNK)
        def _():
            pltpu.make_async_copy(pe_hbm.at[orders_v.at[pl.ds(0, CHUNK)]], rbuf1, sem1).wait()
            pltpu.sync_copy(rbuf1, out_hbm.at[pl.ds(out0 + (c + 1) * CHUNK, CHUNK)])

        return carry

    lax.fori_loop(0, (NCHUNK + 1) // 2, chunk_step, jnp.int32(0))
    del cp


def kernel(visit_concept_orders, pe):
    mesh = plsc.VectorSubcoreMesh(core_axis_name="c", subcore_axis_name="s")
    run = functools.partial(
        pl.kernel,
        mesh=mesh,
        compiler_params=pltpu.CompilerParams(
            needs_layout_passes=False, use_tc_tiling_on_sc=False
        ),
        out_type=jax.ShapeDtypeStruct((B * L, EMB), jnp.float32),
        scratch_types=[
            pltpu.VMEM((FLAT_PW,), jnp.int32),
            pltpu.VMEM((FLAT_PW,), jnp.int32),
            pltpu.VMEM((CHUNK, EMB), jnp.float32),
            pltpu.VMEM((CHUNK, EMB), jnp.float32),
            pltpu.SemaphoreType.DMA,
            pltpu.SemaphoreType.DMA,
        ],
    )(_body)
    out = run(visit_concept_orders.reshape(B * L), pe)
    return out.reshape(B, L, EMB)


# speedup vs baseline: 3.3593x; 1.0135x over previous
import functools

import jax
import jax.numpy as jnp
from jax import lax
from jax.experimental import pallas as pl
from jax.experimental.pallas import tpu as pltpu
from jax.experimental.pallas import tpu_sc as plsc

LARGE_POSITION_VALUE = 1000000
MAX_SEQ_LEN = 512
EMB = 64
B, L = 4096, 200
NW = 32
RPW = B // NW
FLAT_PW = RPW * L
CHUNK = 128
NCHUNK = FLAT_PW // CHUNK
GROUPS = RPW // 16


NBUF = 8
AHEAD = 4
UNROLL = 4


def _body(vco_hbm, pe_hbm, out_hbm, vco_v, rbuf, *sems):
    gsem = sems[:NBUF]
    osem = sems[NBUF:]
    cid = lax.axis_index("c")
    sid = lax.axis_index("s")
    wid = sid * 2 + cid
    out0 = wid * FLAT_PW

    pltpu.sync_copy(vco_hbm.at[pl.ds(wid * FLAT_PW, FLAT_PW)], vco_v)

    lanes = lax.broadcasted_iota(jnp.int32, (16,), 0)
    for g in range(GROUPS):
        rows = lanes + g * 16
        flat_base = rows * L

        def min_step(i, m):
            for k in range(UNROLL):
                col = plsc.load_gather(vco_v, [flat_base + (i * UNROLL + k)])
                m = jnp.minimum(m, jnp.where(col > 0, col, LARGE_POSITION_VALUE))
            return m

        m = lax.fori_loop(
            0, L // UNROLL, min_step,
            jnp.full((16,), LARGE_POSITION_VALUE, jnp.int32),
        )

        def ord_step(i, carry):
            for k in range(UNROLL):
                idx = flat_base + (i * UNROLL + k)
                col = plsc.load_gather(vco_v, [idx])
                o = jnp.minimum(jnp.maximum(col - m, 0), jnp.int32(MAX_SEQ_LEN - 1))
                plsc.store_scatter(vco_v, [idx], o)
            return carry

        lax.fori_loop(0, L // UNROLL, ord_step, jnp.int32(0))

    def gather(c, s):
        pltpu.async_copy(
            pe_hbm.at[vco_v.at[pl.ds(c * CHUNK, CHUNK)]], rbuf.at[s], gsem[s]
        )

    def wait_gather(s):
        pltpu.make_async_copy(pe_hbm.at[vco_v.at[pl.ds(0, CHUNK)]], rbuf.at[s], gsem[s]).wait()

    def out_start(c, s):
        pltpu.make_async_copy(
            rbuf.at[s], out_hbm.at[pl.ds(out0 + c * CHUNK, CHUNK)], osem[s]
        ).start()

    def wait_out(c, s):
        pltpu.make_async_copy(
            rbuf.at[s], out_hbm.at[pl.ds(out0 + c * CHUNK, CHUNK)], osem[s]
        ).wait()

    for c in range(AHEAD):
        gather(c, c % NBUF)

    def chunk_step(i, carry):
        for s8 in range(NBUF):
            c = i * NBUF + s8
            wait_gather(s8)
            out_start(c, s8)
            nxt = (s8 + AHEAD) % NBUF

            @pl.when(c >= AHEAD)
            def _():
                wait_out(c - AHEAD, nxt)

            @pl.when(c + AHEAD < NCHUNK)
            def _():
                gather(c + AHEAD, nxt)

        return carry

    lax.fori_loop(0, NCHUNK // NBUF, chunk_step, jnp.int32(0))
    for c in range(NCHUNK - AHEAD, NCHUNK):
        wait_out(c, c % NBUF)


def kernel(visit_concept_orders, pe):
    mesh = plsc.VectorSubcoreMesh(core_axis_name="c", subcore_axis_name="s")
    run = functools.partial(
        pl.kernel,
        mesh=mesh,
        compiler_params=pltpu.CompilerParams(
            needs_layout_passes=False, use_tc_tiling_on_sc=False
        ),
        out_type=jax.ShapeDtypeStruct((B * L, EMB), jnp.float32),
        scratch_types=[
            pltpu.VMEM((FLAT_PW,), jnp.int32),
            pltpu.VMEM((NBUF, CHUNK, EMB), jnp.float32),
        ] + [pltpu.SemaphoreType.DMA] * (2 * NBUF),
    )(_body)
    out = run(visit_concept_orders.reshape(B * L), pe)
    return out.reshape(B, L, EMB)
